# baseline (speedup 1.0000x reference)
import functools

import jax
import jax.numpy as jnp
from jax import lax
from jax.experimental import pallas as pl
from jax.experimental.pallas import tpu as pltpu
from jax.experimental.pallas import tpu_sc as plsc

_DIM = 300
_DIMP = 384
_HIST = 50
_HISTP = 56
_NC = 2
_NS = 16
_NW = _NC * _NS


@functools.lru_cache(maxsize=None)
def _make_kernel(S):
    assert S % _NW == 0
    s_per_w = S // _NW
    assert s_per_w % 2 == 0
    mesh = plsc.VectorSubcoreMesh(core_axis_name="c", subcore_axis_name="s")

    @functools.partial(
        pl.kernel,
        mesh=mesh,
        out_type=jax.ShapeDtypeStruct((S, _HIST, _DIM), jnp.float32),
        scratch_types=[
            pltpu.VMEM((s_per_w * _HISTP,), jnp.int32),
            pltpu.VMEM((_HISTP, _DIMP), jnp.float32),
            pltpu.VMEM((_HISTP, _DIMP), jnp.float32),
            pltpu.VMEM((_HIST, _DIM), jnp.float32),
            pltpu.VMEM((_HIST, _DIM), jnp.float32),
            pltpu.SemaphoreType.DMA,
            pltpu.SemaphoreType.DMA,
            pltpu.SemaphoreType.DMA,
            pltpu.SemaphoreType.DMA,
        ],
    )
    def gather(idx_hbm, table_hbm, out_hbm, idx_v, rows0, rows1,
               til0, til1, gsem0, gsem1, osem0, osem1):
        wid = lax.axis_index("s") * _NC + lax.axis_index("c")
        sbase = wid * s_per_w
        pltpu.sync_copy(
            idx_hbm.at[pl.ds(sbase * _HISTP, s_per_w * _HISTP)], idx_v)
        rows = (rows0, rows1)
        til = (til0, til1)
        gsems = (gsem0, gsem1)
        osems = (osem0, osem1)

        def start_gather(j, b):
            pltpu.async_copy(
                table_hbm.at[idx_v.at[pl.ds(j * _HISTP, _HISTP)]],
                rows[b], gsems[b])

        def wait_gather(b):
            pltpu.make_async_copy(
                table_hbm.at[idx_v.at[pl.ds(0, _HISTP)]], rows[b], gsems[b]
            ).wait()

        def start_out(j, b):
            pltpu.async_copy(til[b], out_hbm.at[sbase + j], osems[b])

        def wait_out(b):
            pltpu.make_async_copy(til[b], out_hbm.at[sbase], osems[b]).wait()

        start_gather(0, 0)
        start_gather(1, 1)

        def body(jp, carry):
            for b in range(2):
                j = jp * 2 + b
                wait_gather(b)

                @pl.when(j >= 2)
                def _():
                    wait_out(b)

                def row_body(r, c):
                    for k in range(_DIM // 16):
                        til[b][r, pl.ds(16 * k, 16)] = rows[b][r, pl.ds(16 * k, 16)]
                    til[b][r, pl.ds(284, 16)] = rows[b][r, pl.ds(284, 16)]
                    return c

                lax.fori_loop(0, _HIST, row_body, 0)

                @pl.when(j + 2 < s_per_w)
                def _():
                    start_gather(j + 2, b)

                start_out(j, b)
            return carry

        lax.fori_loop(0, s_per_w // 2, body, 0)
        wait_out(0)
        wait_out(1)

    return gather


def kernel(x, table):
    S, H = x.shape
    xi = jnp.pad(x.astype(jnp.int32), ((0, 0), (0, _HISTP - H))).reshape(-1)
    tpad = jnp.pad(table, ((0, 0), (0, _DIMP - table.shape[1])))
    return _make_kernel(S)(xi, tpad)
